# baseline (speedup 1.0000x reference)
import jax
import jax.numpy as jnp
from jax import lax
from jax.experimental import pallas as pl
from jax.experimental.pallas import tpu as pltpu
from jax.experimental.pallas import tpu_sc as plsc

B, L, D = 4, 4096, 1024
NC, NS, NL = 2, 16, 16
NW = NC * NS
LW = L // NW
R = 16
NCHUNK = LW // R
CH = R * D
STEPS = NCHUNK * B
NXB = 4
GROUP = 2 * B
NGROUP = STEPS // GROUP


def _body(x_hbm, tbl_hbm, out_hbm, tb0, tb1, xb0, xb1, xb2, xb3,
          sems_t, sems_x, sems_o):
    tbufs = [tb0, tb1]
    xbufs = [xb0, xb1, xb2, xb3]
    wid = lax.axis_index("s") * NC + lax.axis_index("c")
    lbase = wid * LW

    def start_t(c, slot):
        return pltpu.async_copy(
            tbl_hbm.at[pl.ds(lbase + c * R, R), :], tbufs[slot],
            sems_t.at[slot])

    def x_slice(g):
        c = g // B
        b = g % B
        return x_hbm.at[b, pl.ds(lbase + c * R, R), :]

    def out_slice(g):
        c = g // B
        b = g % B
        return out_hbm.at[b, pl.ds(lbase + c * R, R), :]

    def start_x(g, slot):
        return pltpu.async_copy(x_slice(g), xbufs[slot], sems_x.at[slot])

    def start_o(g, slot):
        return pltpu.async_copy(xbufs[slot], out_slice(g), sems_o.at[slot])

    def wait_x(g, slot):
        pltpu.make_async_copy(x_slice(g), xbufs[slot], sems_x.at[slot]).wait()

    def wait_o(g, slot):
        pltpu.make_async_copy(xbufs[slot], out_slice(g), sems_o.at[slot]).wait()

    def wait_t(c, slot):
        pltpu.make_async_copy(
            tbl_hbm.at[pl.ds(lbase + c * R, R), :], tbufs[slot],
            sems_t.at[slot]).wait()

    start_t(0, 0)
    for p in range(NXB - 1):
        start_x(p, p)

    def group(m, carry):
        for sl in range(GROUP):
            g = m * GROUP + sl
            c = 2 * m + sl // B
            tslot = (sl // B) % 2
            if sl == 0:
                wait_t(c, 0)
                start_t(c + 1, 1)
            if sl == B:
                wait_t(c, 1)

                @pl.when(c + 1 < NCHUNK)
                def _():
                    start_t(c + 1, 0)
            xslot = sl % NXB
            wait_x(g, xslot)
            tbuf = tbufs[tslot]
            xbuf = xbufs[xslot]

            @plsc.parallel_loop(0, CH, NL, unroll=2)
            def _(o):
                i = o // D
                j = o % D
                plsc.addupdate(xbuf.at[i, pl.ds(j, NL)], tbuf[i, pl.ds(j, NL)])

            start_o(g, xslot)
            pslot = (sl + NXB - 1) % NXB
            if sl == 0:
                @pl.when(g >= 1)
                def _():
                    wait_o(g - 1, pslot)

                @pl.when(g + NXB - 1 < STEPS)
                def _():
                    start_x(g + NXB - 1, pslot)
            else:
                @pl.when(g + NXB - 1 < STEPS)
                def _():
                    wait_o(g - 1, pslot)
                    start_x(g + NXB - 1, pslot)
        return carry

    lax.fori_loop(0, NGROUP, group, None)
    for k in range(NXB, 0, -1):
        g = STEPS - k
        wait_o(g, g % NXB)


@jax.jit
def _run(x, tbl):
    mesh = plsc.VectorSubcoreMesh(core_axis_name="c", subcore_axis_name="s")
    return pl.kernel(
        _body,
        out_type=jax.ShapeDtypeStruct((B, L, D), jnp.float32),
        mesh=mesh,
        scratch_types=[
            pltpu.VMEM((R, D), jnp.float32),
            pltpu.VMEM((R, D), jnp.float32),
            pltpu.VMEM((R, D), jnp.float32),
            pltpu.VMEM((R, D), jnp.float32),
            pltpu.VMEM((R, D), jnp.float32),
            pltpu.VMEM((R, D), jnp.float32),
            pltpu.SemaphoreType.DMA((2,)),
            pltpu.SemaphoreType.DMA((NXB,)),
            pltpu.SemaphoreType.DMA((NXB,)),
        ],
    )(x, tbl)


def kernel(x, pos_emb_table):
    return _run(x, pos_emb_table)


# speedup vs baseline: 1.2201x; 1.2201x over previous
import jax
import jax.numpy as jnp
from jax import lax
from jax.experimental import pallas as pl
from jax.experimental.pallas import tpu as pltpu
from jax.experimental.pallas import tpu_sc as plsc

B, L, D = 4, 4096, 1024
NC, NS, NL = 2, 16, 16
NW = NC * NS
LW = L // NW
R = 16
NCHUNK = LW // R
CH = R * D
STEPS = NCHUNK * B
NXB = 4
GROUP = 2 * B
NGROUP = STEPS // GROUP


def _body(x_hbm, tbl_hbm, out_hbm, tb0, tb1, xb0, xb1, xb2, xb3,
          sems_t, sems_x, sems_o):
    tbufs = [tb0, tb1]
    xbufs = [xb0, xb1, xb2, xb3]
    wid = lax.axis_index("s") * NC + lax.axis_index("c")
    lbase = wid * LW

    def start_t(c, slot):
        return pltpu.async_copy(
            tbl_hbm.at[pl.ds(lbase + c * R, R), :], tbufs[slot],
            sems_t.at[slot])

    def x_slice(g):
        c = g // B
        b = g % B
        return x_hbm.at[b, pl.ds(lbase + c * R, R), :]

    def out_slice(g):
        c = g // B
        b = g % B
        return out_hbm.at[b, pl.ds(lbase + c * R, R), :]

    def start_x(g, slot):
        return pltpu.async_copy(x_slice(g), xbufs[slot], sems_x.at[slot])

    def start_o(g, slot):
        return pltpu.async_copy(xbufs[slot], out_slice(g), sems_o.at[slot])

    def wait_x(g, slot):
        pltpu.make_async_copy(x_slice(g), xbufs[slot], sems_x.at[slot]).wait()

    def wait_o(g, slot):
        pltpu.make_async_copy(xbufs[slot], out_slice(g), sems_o.at[slot]).wait()

    def wait_t(c, slot):
        pltpu.make_async_copy(
            tbl_hbm.at[pl.ds(lbase + c * R, R), :], tbufs[slot],
            sems_t.at[slot]).wait()

    start_t(0, 0)
    for p in range(NXB - 1):
        start_x(p, p)

    def group(m, carry):
        for sl in range(GROUP):
            g = m * GROUP + sl
            c = 2 * m + sl // B
            tslot = (sl // B) % 2
            if sl == 0:
                wait_t(c, 0)
                start_t(c + 1, 1)
            if sl == B:
                wait_t(c, 1)

                @pl.when(c + 1 < NCHUNK)
                def _():
                    start_t(c + 1, 0)
            xslot = sl % NXB
            wait_x(g, xslot)
            tbuf = tbufs[tslot]
            xbuf = xbufs[xslot]

            @plsc.parallel_loop(0, CH, NL, unroll=4)
            def _(o):
                i = o // D
                j = o % D
                plsc.addupdate(xbuf.at[i, pl.ds(j, NL)], tbuf[i, pl.ds(j, NL)])

            start_o(g, xslot)
            pslot = (sl + NXB - 1) % NXB
            if sl == 0:
                @pl.when(g >= 1)
                def _():
                    wait_o(g - 1, pslot)

                @pl.when(g + NXB - 1 < STEPS)
                def _():
                    start_x(g + NXB - 1, pslot)
            else:
                @pl.when(g + NXB - 1 < STEPS)
                def _():
                    wait_o(g - 1, pslot)
                    start_x(g + NXB - 1, pslot)
        return carry

    lax.fori_loop(0, NGROUP, group, None)
    for k in range(NXB, 0, -1):
        g = STEPS - k
        wait_o(g, g % NXB)


@jax.jit
def _run(x, tbl):
    mesh = plsc.VectorSubcoreMesh(core_axis_name="c", subcore_axis_name="s")
    return pl.kernel(
        _body,
        out_type=jax.ShapeDtypeStruct((B, L, D), jnp.float32),
        mesh=mesh,
        scratch_types=[
            pltpu.VMEM((R, D), jnp.float32),
            pltpu.VMEM((R, D), jnp.float32),
            pltpu.VMEM((R, D), jnp.float32),
            pltpu.VMEM((R, D), jnp.float32),
            pltpu.VMEM((R, D), jnp.float32),
            pltpu.VMEM((R, D), jnp.float32),
            pltpu.SemaphoreType.DMA((2,)),
            pltpu.SemaphoreType.DMA((NXB,)),
            pltpu.SemaphoreType.DMA((NXB,)),
        ],
    )(x, tbl)


def kernel(x, pos_emb_table):
    return _run(x, pos_emb_table)


# speedup vs baseline: 1.2360x; 1.0130x over previous
import jax
import jax.numpy as jnp
from jax import lax
from jax.experimental import pallas as pl
from jax.experimental.pallas import tpu as pltpu
from jax.experimental.pallas import tpu_sc as plsc

B, L, D = 4, 4096, 1024
NC, NS, NL = 2, 16, 16
NW = NC * NS
LW = L // NW
R = 8
NCHUNK = LW // R
CH = R * D
STEPS = NCHUNK * B
NXB = 8
GROUP = 2 * B
NGROUP = STEPS // GROUP


def _body(x_hbm, tbl_hbm, out_hbm, tb0, tb1, xb0, xb1, xb2, xb3, xb4, xb5, xb6, xb7,
          sems_t, sems_x, sems_o):
    tbufs = [tb0, tb1]
    xbufs = [xb0, xb1, xb2, xb3, xb4, xb5, xb6, xb7]
    wid = lax.axis_index("s") * NC + lax.axis_index("c")
    lbase = wid * LW

    def start_t(c, slot):
        return pltpu.async_copy(
            tbl_hbm.at[pl.ds(lbase + c * R, R), :], tbufs[slot],
            sems_t.at[slot])

    def x_slice(g):
        c = g // B
        b = g % B
        return x_hbm.at[b, pl.ds(lbase + c * R, R), :]

    def out_slice(g):
        c = g // B
        b = g % B
        return out_hbm.at[b, pl.ds(lbase + c * R, R), :]

    def start_x(g, slot):
        return pltpu.async_copy(x_slice(g), xbufs[slot], sems_x.at[slot])

    def start_o(g, slot):
        return pltpu.async_copy(xbufs[slot], out_slice(g), sems_o.at[slot])

    def wait_x(g, slot):
        pltpu.make_async_copy(x_slice(g), xbufs[slot], sems_x.at[slot]).wait()

    def wait_o(g, slot):
        pltpu.make_async_copy(xbufs[slot], out_slice(g), sems_o.at[slot]).wait()

    def wait_t(c, slot):
        pltpu.make_async_copy(
            tbl_hbm.at[pl.ds(lbase + c * R, R), :], tbufs[slot],
            sems_t.at[slot]).wait()

    start_t(0, 0)
    for p in range(NXB - 1):
        start_x(p, p)

    def group(m, carry):
        for sl in range(GROUP):
            g = m * GROUP + sl
            c = 2 * m + sl // B
            tslot = (sl // B) % 2
            if sl == 0:
                wait_t(c, 0)
                start_t(c + 1, 1)
            if sl == B:
                wait_t(c, 1)

                @pl.when(c + 1 < NCHUNK)
                def _():
                    start_t(c + 1, 0)
            xslot = sl % NXB
            wait_x(g, xslot)
            tbuf = tbufs[tslot]
            xbuf = xbufs[xslot]

            @plsc.parallel_loop(0, CH, NL, unroll=4)
            def _(o):
                i = o // D
                j = o % D
                plsc.addupdate(xbuf.at[i, pl.ds(j, NL)], tbuf[i, pl.ds(j, NL)])

            start_o(g, xslot)
            pslot = (sl + NXB - 1) % NXB
            if sl == 0:
                @pl.when(g >= 1)
                def _():
                    wait_o(g - 1, pslot)

                @pl.when(g + NXB - 1 < STEPS)
                def _():
                    start_x(g + NXB - 1, pslot)
            else:
                @pl.when(g + NXB - 1 < STEPS)
                def _():
                    wait_o(g - 1, pslot)
                    start_x(g + NXB - 1, pslot)
        return carry

    lax.fori_loop(0, NGROUP, group, None)
    for k in range(NXB, 0, -1):
        g = STEPS - k
        wait_o(g, g % NXB)


@jax.jit
def _run(x, tbl):
    mesh = plsc.VectorSubcoreMesh(core_axis_name="c", subcore_axis_name="s")
    return pl.kernel(
        _body,
        out_type=jax.ShapeDtypeStruct((B, L, D), jnp.float32),
        mesh=mesh,
        scratch_types=[
            pltpu.VMEM((R, D), jnp.float32),
            pltpu.VMEM((R, D), jnp.float32),
            pltpu.VMEM((R, D), jnp.float32),
            pltpu.VMEM((R, D), jnp.float32),
            pltpu.VMEM((R, D), jnp.float32),
            pltpu.VMEM((R, D), jnp.float32),
            pltpu.VMEM((R, D), jnp.float32),
            pltpu.VMEM((R, D), jnp.float32),
            pltpu.VMEM((R, D), jnp.float32),
            pltpu.VMEM((R, D), jnp.float32),
            pltpu.SemaphoreType.DMA((2,)),
            pltpu.SemaphoreType.DMA((NXB,)),
            pltpu.SemaphoreType.DMA((NXB,)),
        ],
    )(x, tbl)


def kernel(x, pos_emb_table):
    return _run(x, pos_emb_table)


# speedup vs baseline: 1.2503x; 1.0116x over previous
import jax
import jax.numpy as jnp
from jax import lax
from jax.experimental import pallas as pl
from jax.experimental.pallas import tpu as pltpu
from jax.experimental.pallas import tpu_sc as plsc

B, L, D = 4, 4096, 1024
NC, NS, NL = 2, 16, 16
NW = NC * NS
LW = L // NW
R = 8
NCHUNK = LW // R
CH = R * D
STEPS = NCHUNK * B
NXB = 8
GROUP = 2 * B
NGROUP = STEPS // GROUP


def _body(x_hbm, tbl_hbm, out_hbm, tb0, tb1, xb0, xb1, xb2, xb3, xb4, xb5, xb6, xb7,
          sems_t, sems_x, sems_o):
    tbufs = [tb0, tb1]
    xbufs = [xb0, xb1, xb2, xb3, xb4, xb5, xb6, xb7]
    wid = lax.axis_index("s") * NC + lax.axis_index("c")
    lbase = wid * LW

    def start_t(c, slot):
        return pltpu.async_copy(
            tbl_hbm.at[pl.ds(lbase + c * R, R), :], tbufs[slot],
            sems_t.at[slot])

    def x_slice(g):
        c = g // B
        b = g % B
        return x_hbm.at[b, pl.ds(lbase + c * R, R), :]

    def out_slice(g):
        c = g // B
        b = g % B
        return out_hbm.at[b, pl.ds(lbase + c * R, R), :]

    def start_x(g, slot):
        return pltpu.async_copy(x_slice(g), xbufs[slot], sems_x.at[slot])

    def start_o(g, slot):
        return pltpu.async_copy(xbufs[slot], out_slice(g), sems_o.at[slot])

    def wait_x(g, slot):
        pltpu.make_async_copy(x_slice(g), xbufs[slot], sems_x.at[slot]).wait()

    def wait_o(g, slot):
        pltpu.make_async_copy(xbufs[slot], out_slice(g), sems_o.at[slot]).wait()

    def wait_t(c, slot):
        pltpu.make_async_copy(
            tbl_hbm.at[pl.ds(lbase + c * R, R), :], tbufs[slot],
            sems_t.at[slot]).wait()

    start_t(0, 0)
    for p in range(NXB - 1):
        start_x(p, p)

    def group(m, carry):
        for sl in range(GROUP):
            g = m * GROUP + sl
            c = 2 * m + sl // B
            tslot = (sl // B) % 2
            if sl == 0:
                wait_t(c, 0)
                start_t(c + 1, 1)
            if sl == B:
                wait_t(c, 1)

                @pl.when(c + 1 < NCHUNK)
                def _():
                    start_t(c + 1, 0)
            xslot = sl % NXB
            wait_x(g, xslot)
            tbuf = tbufs[tslot]
            xbuf = xbufs[xslot]

            @plsc.parallel_loop(0, CH, NL, unroll=8)
            def _(o):
                i = o // D
                j = o % D
                plsc.addupdate(xbuf.at[i, pl.ds(j, NL)], tbuf[i, pl.ds(j, NL)])

            start_o(g, xslot)
            pslot = (sl + NXB - 1) % NXB
            if sl == 0:
                @pl.when(g >= 1)
                def _():
                    wait_o(g - 1, pslot)

                @pl.when(g + NXB - 1 < STEPS)
                def _():
                    start_x(g + NXB - 1, pslot)
            else:
                @pl.when(g + NXB - 1 < STEPS)
                def _():
                    wait_o(g - 1, pslot)
                    start_x(g + NXB - 1, pslot)
        return carry

    lax.fori_loop(0, NGROUP, group, None)
    for k in range(NXB, 0, -1):
        g = STEPS - k
        wait_o(g, g % NXB)


@jax.jit
def _run(x, tbl):
    mesh = plsc.VectorSubcoreMesh(core_axis_name="c", subcore_axis_name="s")
    return pl.kernel(
        _body,
        out_type=jax.ShapeDtypeStruct((B, L, D), jnp.float32),
        mesh=mesh,
        scratch_types=[
            pltpu.VMEM((R, D), jnp.float32),
            pltpu.VMEM((R, D), jnp.float32),
            pltpu.VMEM((R, D), jnp.float32),
            pltpu.VMEM((R, D), jnp.float32),
            pltpu.VMEM((R, D), jnp.float32),
            pltpu.VMEM((R, D), jnp.float32),
            pltpu.VMEM((R, D), jnp.float32),
            pltpu.VMEM((R, D), jnp.float32),
            pltpu.VMEM((R, D), jnp.float32),
            pltpu.VMEM((R, D), jnp.float32),
            pltpu.SemaphoreType.DMA((2,)),
            pltpu.SemaphoreType.DMA((NXB,)),
            pltpu.SemaphoreType.DMA((NXB,)),
        ],
    )(x, tbl)


def kernel(x, pos_emb_table):
    return _run(x, pos_emb_table)
